# baseline (speedup 1.0000x reference)
import functools
import math

import jax
import jax.numpy as jnp
from jax import lax
from jax.experimental import pallas as pl
from jax.experimental.pallas import tpu as pltpu
from jax.experimental.pallas import tpu_sc as plsc

SCALE = 30.0
MARGIN = 0.35
LANES = 128
K2 = SCALE / math.log(2.0)


def _gather_body(C, b_per_w, n_cores, xflat, tgt, xt_out, tgt_v, idx_v,
                 val_v, sem):
    wid = lax.axis_index("s") * n_cores + lax.axis_index("c")
    base = wid * b_per_w
    pltpu.sync_copy(tgt.at[pl.ds(base, b_per_w)], tgt_v)
    for h in range(b_per_w // 16):
        t = tgt_v[pl.ds(h * 16, 16)]
        row = base + h * 16 + lax.iota(jnp.int32, 16)
        idx_v[pl.ds(h * 16, 16)] = row * C + t
    pltpu.async_copy(xflat.at[idx_v], val_v, sem).wait()
    pltpu.sync_copy(val_v, xt_out.at[pl.ds(base, b_per_w)])


def _lmcl_body(C, W, B, x_ref, xt_ref, o_ref, m_scr, s_scr):
    j = pl.program_id(0)
    nj = pl.num_programs(0)
    nch = W // LANES

    @pl.when(j == 0)
    def _init():
        m_scr[...] = jnp.full((B, LANES), -jnp.inf, jnp.float32)
        s_scr[...] = jnp.zeros((B, LANES), jnp.float32)

    x = x_ref[...]
    chunks = [x[:, k * LANES:(k + 1) * LANES] for k in range(nch)]

    def accumulate(cs, masks):
        bm = functools.reduce(
            jnp.maximum,
            [c if mk is None else jnp.where(mk, c, -jnp.inf)
             for c, mk in zip(cs, masks)],
        )
        m_old = m_scr[...]
        m_new = jnp.maximum(m_old, bm)
        mk2 = m_new * K2
        acc = s_scr[...] * jnp.where(
            m_old == -jnp.inf, 0.0, jnp.exp2(m_old * K2 - mk2)
        )
        for c, mk in zip(cs, masks):
            p = jnp.exp2(c * K2 - mk2)
            acc += p if mk is None else jnp.where(mk, p, 0.0)
        m_scr[...] = m_new
        s_scr[...] = acc

    @pl.when(j < nj - 1)
    def _full():
        accumulate(chunks, [None] * nch)

    @pl.when(j == nj - 1)
    def _last():
        last_valid = C - (nj - 1) * W
        lane = lax.broadcasted_iota(jnp.int32, (B, LANES), 1)
        lchunks, lmasks = [], []
        for k in range(nch):
            base = k * LANES
            if base >= last_valid:
                continue
            lchunks.append(chunks[k])
            lmasks.append(
                None if base + LANES <= last_valid
                else lane < last_valid - base
            )
        accumulate(lchunks, lmasks)

        m128 = m_scr[...]
        s128 = s_scr[...]
        m = jnp.max(m128, axis=1, keepdims=True)
        mk2 = m * K2
        s = jnp.sum(s128 * jnp.exp2(m128 * K2 - mk2), axis=1, keepdims=True)
        xt = xt_ref[...]
        s_corr = (s - jnp.exp2(xt * K2 - mk2)
                  + jnp.exp2((xt - MARGIN) * K2 - mk2))
        nll = m * SCALE + jnp.log(s_corr) - (xt - MARGIN) * SCALE
        o_ref[...] = jnp.sum(nll, axis=0, keepdims=True) / B


def _sc_gather(output, tgt_i32):
    B, C = output.shape
    info = plsc.get_sparse_core_info()
    n_workers = info.num_cores * info.num_subcores
    b_per_w = B // n_workers
    mesh = plsc.VectorSubcoreMesh(core_axis_name="c", subcore_axis_name="s")
    return pl.kernel(
        functools.partial(_gather_body, C, b_per_w, info.num_cores),
        mesh=mesh,
        out_type=jax.ShapeDtypeStruct((B,), jnp.float32),
        scratch_types=[
            pltpu.VMEM((b_per_w,), jnp.int32),
            pltpu.VMEM((b_per_w,), jnp.int32),
            pltpu.VMEM((b_per_w,), jnp.float32),
            pltpu.SemaphoreType.DMA,
        ],
    )(output.reshape(-1), tgt_i32)


def kernel(output, target):
    B, C = output.shape
    W = 2048
    nj = pl.cdiv(C, W)
    tgt = target.astype(jnp.int32)

    xt = _sc_gather(output, tgt).reshape(B, 1)

    out = pl.pallas_call(
        functools.partial(_lmcl_body, C, W, B),
        grid=(nj,),
        in_specs=[
            pl.BlockSpec((B, W), lambda j: (0, j)),
            pl.BlockSpec((B, 1), lambda j: (0, 0)),
        ],
        out_specs=pl.BlockSpec((1, 1), lambda j: (0, 0)),
        out_shape=jax.ShapeDtypeStruct((1, 1), jnp.float32),
        scratch_shapes=[
            pltpu.VMEM((B, LANES), jnp.float32),
            pltpu.VMEM((B, LANES), jnp.float32),
        ],
    )(output, xt)
    return out[0, 0]


# speedup vs baseline: 1.2742x; 1.2742x over previous
import functools
import math

import jax
import jax.numpy as jnp
from jax import lax
from jax.experimental import pallas as pl
from jax.experimental.pallas import tpu as pltpu

SCALE = 30.0
MARGIN = 0.35
LANES = 128
ROWS = 8
K2 = SCALE / math.log(2.0)


def _tree(op, xs):
    while len(xs) > 1:
        nxt = [op(xs[i], xs[i + 1]) for i in range(0, len(xs) - 1, 2)]
        if len(xs) % 2:
            nxt.append(xs[-1])
        xs = nxt
    return xs[0]


def _lmcl_body(C, W, B, x_ref, tgt_ref, o_ref, m_scr, s_scr, xt_scr):
    j = pl.program_id(0)
    nj = pl.num_programs(0)
    nch = W // LANES

    @pl.when(j == 0)
    def _init():
        m_scr[...] = jnp.full((B, LANES), -jnp.inf, jnp.float32)
        s_scr[...] = jnp.zeros((B, LANES), jnp.float32)
        xt_scr[...] = jnp.zeros((B, LANES), jnp.float32)

    lane = lax.broadcasted_iota(jnp.int32, (ROWS, LANES), 1)

    def tile_body(chunk_ids, lims, r, carry):
        r8 = pl.multiple_of(r * ROWS, ROWS)
        rows = pl.ds(r8, ROWS)
        tloc = jnp.broadcast_to(tgt_ref[rows, :] - j * W, (ROWS, LANES))
        m_old = m_scr[rows, :]
        xt = xt_scr[rows, :]
        cs = []
        for k, lim in zip(chunk_ids, lims):
            c = x_ref[rows, k * LANES:(k + 1) * LANES]
            eq = (lane + k * LANES) == tloc
            xt = jnp.where(eq, c, xt)
            cs.append((c, lim))
        xt_scr[rows, :] = xt
        bm = _tree(
            jnp.maximum,
            [c if lim is None else jnp.where(lane < lim, c, -jnp.inf)
             for c, lim in cs],
        )
        m_new = jnp.maximum(m_old, bm)
        mk2 = m_new * K2
        resc = jnp.where(m_old == -jnp.inf, 0.0, jnp.exp2(m_old * K2 - mk2))
        ps = []
        for c, lim in cs:
            p = jnp.exp2(c * K2 - mk2)
            ps.append(p if lim is None else jnp.where(lane < lim, p, 0.0))
        m_scr[rows, :] = m_new
        s_scr[rows, :] = s_scr[rows, :] * resc + _tree(jnp.add, ps)
        return carry

    @pl.when(j < nj - 1)
    def _full():
        lax.fori_loop(
            0, B // ROWS,
            functools.partial(tile_body, list(range(nch)), [None] * nch),
            0,
        )

    @pl.when(j == nj - 1)
    def _last():
        last_valid = C - (nj - 1) * W
        chunk_ids, lims = [], []
        for k in range(nch):
            base = k * LANES
            if base >= last_valid:
                continue
            chunk_ids.append(k)
            lims.append(
                None if base + LANES <= last_valid else last_valid - base
            )
        lax.fori_loop(
            0, B // ROWS, functools.partial(tile_body, chunk_ids, lims), 0
        )

        m128 = m_scr[...]
        s128 = s_scr[...]
        m = jnp.max(m128, axis=1, keepdims=True)
        mk2 = m * K2
        s = jnp.sum(s128 * jnp.exp2(m128 * K2 - mk2), axis=1, keepdims=True)
        xt = jnp.sum(xt_scr[...], axis=1, keepdims=True)
        s_corr = (s - jnp.exp2(xt * K2 - mk2)
                  + jnp.exp2((xt - MARGIN) * K2 - mk2))
        nll = m * SCALE + jnp.log(s_corr) - (xt - MARGIN) * SCALE
        o_ref[...] = jnp.sum(nll, axis=0, keepdims=True) / B


def kernel(output, target):
    B, C = output.shape
    W = 2048
    nj = pl.cdiv(C, W)
    tgt = target.astype(jnp.int32).reshape(B, 1)

    out = pl.pallas_call(
        functools.partial(_lmcl_body, C, W, B),
        grid=(nj,),
        in_specs=[
            pl.BlockSpec((B, W), lambda j: (0, j)),
            pl.BlockSpec((B, 1), lambda j: (0, 0)),
        ],
        out_specs=pl.BlockSpec((1, 1), lambda j: (0, 0)),
        out_shape=jax.ShapeDtypeStruct((1, 1), jnp.float32),
        scratch_shapes=[
            pltpu.VMEM((B, LANES), jnp.float32),
            pltpu.VMEM((B, LANES), jnp.float32),
            pltpu.VMEM((B, LANES), jnp.float32),
        ],
    )(output, tgt)
    return out[0, 0]


# speedup vs baseline: 2.1100x; 1.6559x over previous
import functools
import math

import jax
import jax.numpy as jnp
from jax import lax
from jax.experimental import pallas as pl
from jax.experimental.pallas import tpu as pltpu

SCALE = 30.0
MARGIN = 0.35
K2 = SCALE / math.log(2.0)


def _lmcl_body(C, W, B, x_ref, tgt_ref, o_ref, m_scr, s_scr, xt_scr):
    j = pl.program_id(0)
    nj = pl.num_programs(0)

    @pl.when(j == 0)
    def _init():
        m_scr[...] = jnp.full((B, 1), -jnp.inf, jnp.float32)
        s_scr[...] = jnp.zeros((B, 1), jnp.float32)
        xt_scr[...] = jnp.zeros((B, 1), jnp.float32)

    y = x_ref[...] * K2
    lane = lax.broadcasted_iota(jnp.int32, (B, W), 1)
    tloc = tgt_ref[...] - j * W
    eq = lane == tloc
    xt_scr[...] += jnp.sum(jnp.where(eq, y, 0.0), axis=1, keepdims=True)

    def update(yv):
        m_old = m_scr[...]
        m_new = jnp.maximum(m_old, jnp.max(yv, axis=1, keepdims=True))
        p = jnp.exp2(yv - m_new)
        s_scr[...] = (
            s_scr[...] * jnp.exp2(m_old - m_new)
            + jnp.sum(p, axis=1, keepdims=True)
        )
        m_scr[...] = m_new

    @pl.when(j < nj - 1)
    def _full():
        update(y)

    @pl.when(j == nj - 1)
    def _last():
        last_valid = C - (nj - 1) * W
        update(jnp.where(lane < last_valid, y, -jnp.inf))

        m = m_scr[...]
        s = s_scr[...]
        xt = xt_scr[...]
        mgn = MARGIN * K2
        s_corr = s - jnp.exp2(xt - m) + jnp.exp2(xt - mgn - m)
        nll = (m + jnp.log2(s_corr) - (xt - mgn)) * math.log(2.0)
        o_ref[...] = jnp.sum(nll, axis=0, keepdims=True) / B


def kernel(output, target):
    B, C = output.shape
    W = 2048
    nj = pl.cdiv(C, W)
    tgt = target.astype(jnp.int32).reshape(B, 1)

    out = pl.pallas_call(
        functools.partial(_lmcl_body, C, W, B),
        grid=(nj,),
        in_specs=[
            pl.BlockSpec((B, W), lambda j: (0, j)),
            pl.BlockSpec((B, 1), lambda j: (0, 0)),
        ],
        out_specs=pl.BlockSpec((1, 1), lambda j: (0, 0)),
        out_shape=jax.ShapeDtypeStruct((1, 1), jnp.float32),
        scratch_shapes=[
            pltpu.VMEM((B, 1), jnp.float32),
            pltpu.VMEM((B, 1), jnp.float32),
            pltpu.VMEM((B, 1), jnp.float32),
        ],
    )(output, tgt)
    return out[0, 0]
